# baseline (speedup 1.0000x reference)
import functools

import jax
import jax.numpy as jnp
from jax import lax
from jax.experimental import pallas as pl
from jax.experimental.pallas import tpu as pltpu
from jax.experimental.pallas import tpu_sc as plsc

NUM_CORES = 1
NUM_SUBCORES = 16
NUM_WORKERS = NUM_CORES * NUM_SUBCORES


@functools.lru_cache(maxsize=None)
def _build(n, b, d):
    assert b % (8 * NUM_WORKERS) == 0
    b_per_w = b // NUM_WORKERS
    mesh = plsc.VectorSubcoreMesh(
        core_axis_name="c", subcore_axis_name="s",
        num_cores=NUM_CORES, num_subcores=NUM_SUBCORES)

    row_chunk = min(b_per_w, 256)
    n_chunks = b_per_w // row_chunk

    @functools.partial(
        pl.kernel,
        out_type=jax.ShapeDtypeStruct((b, d), jnp.float32),
        mesh=mesh,
        scratch_types=[
            pltpu.VMEM((b,), jnp.int32),
            pltpu.VMEM((b,), jnp.int32),
            pltpu.VMEM((b_per_w,), jnp.int32),
            pltpu.VMEM((b_per_w,), jnp.int32),
            pltpu.VMEM((row_chunk, d), jnp.float32),
            pltpu.VMEM_SHARED((n,), jnp.int32),
            pltpu.SemaphoreType.DMA,
        ],
    )
    def bank(ids_hbm, iota_hbm, upd_hbm, out_hbm,
             ids_all_v, iota_v, ids_v, w_v, rows_v, table_sh, sem):
        c = lax.axis_index("c")
        s = lax.axis_index("s")
        wid = c * NUM_SUBCORES + s

        @pl.when(s == 0)
        def _phase1():
            pltpu.sync_copy(ids_hbm, ids_all_v)
            pltpu.sync_copy(iota_hbm, iota_v)
            pltpu.sync_copy(iota_v, table_sh.at[ids_all_v])

        plsc.subcore_barrier()

        base = wid * b_per_w
        pltpu.sync_copy(ids_hbm.at[pl.ds(base, b_per_w)], ids_v)
        pltpu.sync_copy(table_sh.at[ids_v], w_v)
        for k in range(n_chunks):
            pltpu.async_copy(
                upd_hbm.at[w_v.at[pl.ds(k * row_chunk, row_chunk)]],
                rows_v, sem).wait()
            pltpu.sync_copy(
                rows_v, out_hbm.at[pl.ds(base + k * row_chunk, row_chunk)])

    return bank


def kernel(node_memories, node_ids, updated_node_memories):
    n = node_memories.shape[0]
    b, d = updated_node_memories.shape
    ids = node_ids.astype(jnp.int32)
    iota = jnp.arange(b, dtype=jnp.int32)
    return _build(n, b, d)(ids, iota, updated_node_memories)


# speedup vs baseline: 1.1275x; 1.1275x over previous
import functools

import jax
import jax.numpy as jnp
from jax import lax
from jax.experimental import pallas as pl
from jax.experimental.pallas import tpu as pltpu
from jax.experimental.pallas import tpu_sc as plsc

NUM_CORES = 2
NUM_SUBCORES = 16
NUM_WORKERS = NUM_CORES * NUM_SUBCORES


@functools.lru_cache(maxsize=None)
def _build(n, b, d):
    assert b % (8 * NUM_WORKERS) == 0
    b_per_w = b // NUM_WORKERS
    mesh = plsc.VectorSubcoreMesh(
        core_axis_name="c", subcore_axis_name="s",
        num_cores=NUM_CORES, num_subcores=NUM_SUBCORES)

    row_chunk = min(b_per_w, 256)
    n_chunks = b_per_w // row_chunk

    @functools.partial(
        pl.kernel,
        out_type=jax.ShapeDtypeStruct((b, d), jnp.float32),
        mesh=mesh,
        scratch_types=[
            pltpu.VMEM((b,), jnp.int32),
            pltpu.VMEM((b,), jnp.int32),
            pltpu.VMEM((b_per_w,), jnp.int32),
            pltpu.VMEM((b_per_w,), jnp.int32),
            pltpu.VMEM((row_chunk, d), jnp.float32),
            pltpu.VMEM_SHARED((n,), jnp.int32),
            pltpu.SemaphoreType.DMA,
        ],
    )
    def bank(ids_hbm, iota_hbm, upd_hbm, out_hbm,
             ids_all_v, iota_v, ids_v, w_v, rows_v, table_sh, sem):
        c = lax.axis_index("c")
        s = lax.axis_index("s")
        wid = c * NUM_SUBCORES + s

        @pl.when(s == 0)
        def _phase1():
            pltpu.sync_copy(ids_hbm, ids_all_v)
            pltpu.sync_copy(iota_hbm, iota_v)
            pltpu.sync_copy(iota_v, table_sh.at[ids_all_v])

        plsc.subcore_barrier()

        base = wid * b_per_w
        pltpu.sync_copy(ids_hbm.at[pl.ds(base, b_per_w)], ids_v)
        pltpu.sync_copy(table_sh.at[ids_v], w_v)
        for k in range(n_chunks):
            pltpu.async_copy(
                upd_hbm.at[w_v.at[pl.ds(k * row_chunk, row_chunk)]],
                rows_v, sem).wait()
            pltpu.sync_copy(
                rows_v, out_hbm.at[pl.ds(base + k * row_chunk, row_chunk)])

    return bank


def kernel(node_memories, node_ids, updated_node_memories):
    n = node_memories.shape[0]
    b, d = updated_node_memories.shape
    ids = node_ids.astype(jnp.int32)
    iota = jnp.arange(b, dtype=jnp.int32)
    return _build(n, b, d)(ids, iota, updated_node_memories)
